# Initial kernel scaffold; baseline (speedup 1.0000x reference)
import jax
import jax.numpy as jnp
from jax.experimental import pallas as pl


def kernel(z, codebook, v):
    raise NotImplementedError("write your pallas kernel here")


# speedup vs baseline: 1.6478x; 1.6478x over previous
import functools

import jax
import jax.numpy as jnp
from jax import lax
from jax.experimental import pallas as pl
from jax.experimental.pallas import tpu as pltpu
from jax.experimental.pallas import tpu_sc as plsc

B = 8192
D = 256
K = 8192

BM = 1024
BK = 1024
NB = B // BM
NK = K // BK


def _argmin_body(z_ref, cb_ref, idx_ref, minv_ref):
    k = pl.program_id(1)

    @pl.when(k == 0)
    def _():
        minv_ref[...] = jnp.full((1, BM), jnp.inf, jnp.float32)
        idx_ref[...] = jnp.zeros((1, 1, BM), jnp.int32)

    zs = z_ref[...] * (-2.0)
    cb = cb_ref[...]
    c_sq = jnp.sum(cb * cb, axis=1, keepdims=True)
    scores = lax.dot_general(cb, zs, (((1,), (1,)), ((), ())),
                             preferred_element_type=jnp.float32) + c_sq
    m = jnp.min(scores, axis=0, keepdims=True)
    rows = lax.broadcasted_iota(jnp.int32, scores.shape, 0) + k * BK
    bidx = jnp.min(jnp.where(scores == m, rows, K), axis=0, keepdims=True)
    better = m < minv_ref[...]
    minv_ref[...] = jnp.where(better, m, minv_ref[...])
    idx_ref[...] = jnp.where(better[None], bidx[None], idx_ref[...])


def _argmin_call(z, cb):
    out = pl.pallas_call(
        _argmin_body,
        grid=(NB, NK),
        in_specs=[pl.BlockSpec((BM, D), lambda i, k: (i, 0)),
                  pl.BlockSpec((BK, D), lambda i, k: (k, 0))],
        out_specs=pl.BlockSpec((1, 1, BM), lambda i, k: (i, 0, 0)),
        out_shape=jax.ShapeDtypeStruct((NB, 1, BM), jnp.int32),
        scratch_shapes=[pltpu.VMEM((1, BM), jnp.float32)],
        compiler_params=pltpu.CompilerParams(
            dimension_semantics=("arbitrary", "arbitrary")),
    )(z, cb)
    return out.reshape(B)


SC_NC = 2
SC_NS = 16
NW = SC_NC * SC_NS
BPW = B // NW
CHK = 128
NCHK = BPW // CHK

@functools.cache
def _build_gather():
    mesh = plsc.VectorSubcoreMesh(core_axis_name="c", subcore_axis_name="s")

    @functools.partial(
        pl.kernel, mesh=mesh,
        out_type=jax.ShapeDtypeStruct((B, D), jnp.float32),
        scratch_types=[
            pltpu.VMEM((NCHK, CHK), jnp.int32),
            pltpu.VMEM((BPW, D), jnp.float32),
            pltpu.SemaphoreType.DMA,
        ],
    )
    def _gather(idx_hbm, table_hbm, out_hbm, idx_v, rows_v, sem):
        wid = lax.axis_index("s") * SC_NC + lax.axis_index("c")
        base = wid * BPW
        pltpu.sync_copy(idx_hbm.at[pl.ds(wid * NCHK, NCHK)], idx_v)
        copies = [
            pltpu.async_copy(table_hbm.at[idx_v.at[j]],
                             rows_v.at[pl.ds(j * CHK, CHK)], sem)
            for j in range(NCHK)
        ]
        for cp in copies:
            cp.wait()
        pltpu.sync_copy(rows_v, out_hbm.at[pl.ds(base, BPW)])

    return _gather


BC = 1024
NBC = B // BC


def _finish_body(z_ref, n_ref, v_ref, zq_ref, dist_ref, acc_ref):
    i = pl.program_id(0)
    z = z_ref[...]
    d = n_ref[...] - z
    d2 = jnp.sum(d * d, axis=1, keepdims=True)
    dist = jnp.sqrt(d2)
    vd = v_ref[...] + d
    vn = jnp.sqrt(jnp.sum(vd * vd, axis=1, keepdims=True)) + 1e-8
    zq_ref[...] = z + vd * (dist / vn)
    dist_ref[...] = dist
    prev = jnp.where(i == 0, 0.0, acc_ref[0, 0])
    acc_ref[...] = jnp.full((1, 1), prev + jnp.sum(d2), jnp.float32)


def _finish_call(z, nearest, v):
    return pl.pallas_call(
        _finish_body,
        grid=(NBC,),
        in_specs=[pl.BlockSpec((BC, D), lambda i: (i, 0)),
                  pl.BlockSpec((BC, D), lambda i: (i, 0)),
                  pl.BlockSpec((BC, D), lambda i: (i, 0))],
        out_specs=[pl.BlockSpec((BC, D), lambda i: (i, 0)),
                   pl.BlockSpec((BC, 1), lambda i: (i, 0)),
                   pl.BlockSpec((1, 1), lambda i: (0, 0))],
        out_shape=[jax.ShapeDtypeStruct((B, D), jnp.float32),
                   jax.ShapeDtypeStruct((B, 1), jnp.float32),
                   jax.ShapeDtypeStruct((1, 1), jnp.float32)],
        compiler_params=pltpu.CompilerParams(
            dimension_semantics=("arbitrary",)),
    )(z, nearest, v)


def kernel(z, codebook, v):
    indices = _argmin_call(z, codebook)
    nearest = _build_gather()(indices.reshape(NW * NCHK, CHK), codebook)
    z_q, dist2d, acc = _finish_call(z, nearest, v)
    dist = dist2d.reshape(B)
    commit_loss = acc[0, 0] / jnp.float32(B * D)
    return (z_q, indices, dist, commit_loss)


# speedup vs baseline: 2.1170x; 1.2848x over previous
import functools

import jax
import jax.numpy as jnp
from jax import lax
from jax.experimental import pallas as pl
from jax.experimental.pallas import tpu as pltpu
from jax.experimental.pallas import tpu_sc as plsc

B = 8192
D = 256
K = 8192

BM = 2048
BK = 1024
NB = B // BM
NK = K // BK


def _argmin_body(z_ref, cb_ref, idx_ref, minv_ref):
    k = pl.program_id(1)

    @pl.when(k == 0)
    def _():
        minv_ref[...] = jnp.full((1, BM), jnp.inf, jnp.float32)
        idx_ref[...] = jnp.zeros((1, 1, BM), jnp.int32)

    zs = z_ref[...] * (-2.0)
    cb = cb_ref[...]
    c_sq = jnp.sum(cb * cb, axis=1, keepdims=True)
    scores = lax.dot_general(cb, zs, (((1,), (1,)), ((), ())),
                             preferred_element_type=jnp.float32) + c_sq
    val = scores
    idx = lax.broadcasted_iota(jnp.int32, scores.shape, 0) + k * BK
    h = BK // 2
    while h >= 8:
        keep = val[:h] <= val[h:]
        val = jnp.where(keep, val[:h], val[h:])
        idx = jnp.where(keep, idx[:h], idx[h:])
        h //= 2
    m = jnp.min(val, axis=0, keepdims=True)
    bidx = jnp.min(jnp.where(val == m, idx, K), axis=0, keepdims=True)
    better = m < minv_ref[...]
    minv_ref[...] = jnp.where(better, m, minv_ref[...])
    idx_ref[...] = jnp.where(better[None], bidx[None], idx_ref[...])


def _argmin_call(z, cb):
    out = pl.pallas_call(
        _argmin_body,
        grid=(NB, NK),
        in_specs=[pl.BlockSpec((BM, D), lambda i, k: (i, 0)),
                  pl.BlockSpec((BK, D), lambda i, k: (k, 0))],
        out_specs=pl.BlockSpec((1, 1, BM), lambda i, k: (i, 0, 0)),
        out_shape=jax.ShapeDtypeStruct((NB, 1, BM), jnp.int32),
        scratch_shapes=[pltpu.VMEM((1, BM), jnp.float32)],
        compiler_params=pltpu.CompilerParams(
            dimension_semantics=("arbitrary", "arbitrary")),
    )(z, cb)
    return out.reshape(B)


SC_NC = 2
SC_NS = 16
NW = SC_NC * SC_NS
BPW = B // NW
CHK = 128
NCHK = BPW // CHK

@functools.cache
def _build_gather():
    mesh = plsc.VectorSubcoreMesh(core_axis_name="c", subcore_axis_name="s")

    @functools.partial(
        pl.kernel, mesh=mesh,
        out_type=jax.ShapeDtypeStruct((B, D), jnp.float32),
        scratch_types=[
            pltpu.VMEM((NCHK, CHK), jnp.int32),
            pltpu.VMEM((BPW, D), jnp.float32),
            pltpu.SemaphoreType.DMA,
        ],
    )
    def _gather(idx_hbm, table_hbm, out_hbm, idx_v, rows_v, sem):
        wid = lax.axis_index("s") * SC_NC + lax.axis_index("c")
        base = wid * BPW
        pltpu.sync_copy(idx_hbm.at[pl.ds(wid * NCHK, NCHK)], idx_v)
        copies = [
            pltpu.async_copy(table_hbm.at[idx_v.at[j]],
                             rows_v.at[pl.ds(j * CHK, CHK)], sem)
            for j in range(NCHK)
        ]
        for cp in copies:
            cp.wait()
        pltpu.sync_copy(rows_v, out_hbm.at[pl.ds(base, BPW)])

    return _gather


BC = 1024
NBC = B // BC


def _finish_body(z_ref, n_ref, v_ref, zq_ref, dist_ref, acc_ref):
    i = pl.program_id(0)
    z = z_ref[...]
    d = n_ref[...] - z
    d2 = jnp.sum(d * d, axis=1, keepdims=True)
    dist = jnp.sqrt(d2)
    vd = v_ref[...] + d
    vn = jnp.sqrt(jnp.sum(vd * vd, axis=1, keepdims=True)) + 1e-8
    zq_ref[...] = z + vd * (dist / vn)
    dist_ref[...] = dist
    prev = jnp.where(i == 0, 0.0, acc_ref[0, 0])
    acc_ref[...] = jnp.full((1, 1), prev + jnp.sum(d2), jnp.float32)


def _finish_call(z, nearest, v):
    return pl.pallas_call(
        _finish_body,
        grid=(NBC,),
        in_specs=[pl.BlockSpec((BC, D), lambda i: (i, 0)),
                  pl.BlockSpec((BC, D), lambda i: (i, 0)),
                  pl.BlockSpec((BC, D), lambda i: (i, 0))],
        out_specs=[pl.BlockSpec((BC, D), lambda i: (i, 0)),
                   pl.BlockSpec((BC, 1), lambda i: (i, 0)),
                   pl.BlockSpec((1, 1), lambda i: (0, 0))],
        out_shape=[jax.ShapeDtypeStruct((B, D), jnp.float32),
                   jax.ShapeDtypeStruct((B, 1), jnp.float32),
                   jax.ShapeDtypeStruct((1, 1), jnp.float32)],
        compiler_params=pltpu.CompilerParams(
            dimension_semantics=("arbitrary",)),
    )(z, nearest, v)


def kernel(z, codebook, v):
    indices = _argmin_call(z, codebook)
    nearest = _build_gather()(indices.reshape(NW * NCHK, CHK), codebook)
    z_q, dist2d, acc = _finish_call(z, nearest, v)
    dist = dist2d.reshape(B)
    commit_loss = acc[0, 0] / jnp.float32(B * D)
    return (z_q, indices, dist, commit_loss)
